# baseline (speedup 1.0000x reference)
import functools

import jax
import jax.numpy as jnp
from jax.experimental import pallas as pl
from jax.experimental.pallas import tpu as pltpu

MIN_TEMPERATURE = 1e-05
NEG = -1e9


def _reduce_kernel(scores_ref, mask_ref, hidden_ref, lnw_ref, lnb_ref,
                   w_ref, bstop_ref, temp_ref,
                   act_ref, lpf_ref, cm_ref, clz_ref, stop_ref, *, n, h,
                   rows):
    i = pl.program_id(0)
    scores = scores_ref[:, :]
    mask = mask_ref[:, :] != 0

    masked = jnp.where(mask, scores, jnp.float32(NEG))
    maxv = jnp.max(masked, axis=1)
    has_edge = maxv != jnp.float32(NEG)
    has_f = has_edge.astype(jnp.float32)
    mes = jnp.where(has_edge, maxv, jnp.float32(0.0))

    lnw = lnw_ref[0, :]
    lnb = lnb_ref[0, :]
    w = w_ref[0, :]
    a_full = lnw * w
    a = a_full[:h]
    s_mes = a_full[h]
    s_he = a_full[h + 1]
    c0 = jnp.sum(lnb * w) + bstop_ref[0, 0]
    t = jnp.maximum(temp_ref[0, 0], jnp.float32(MIN_TEMPERATURE))
    inv_t = 1.0 / t

    hid = hidden_ref[:, :]
    denom = jnp.float32(h + 2)
    mu = (jnp.sum(hid, axis=1) + mes + has_f) / denom
    dh = hid - mu[:, None]
    var = (jnp.sum(dh * dh, axis=1) + jnp.square(mes - mu)
           + jnp.square(has_f - mu)) / denom
    inv_std = jax.lax.rsqrt(var + jnp.float32(1e-5))

    dot = jnp.sum(dh * a[None, :], axis=1)
    stop = inv_std * (dot + (mes - mu) * s_mes + (has_f - mu) * s_he) + c0

    m_all = jnp.maximum(stop, maxv) * inv_t
    z_edges = jnp.sum(jnp.exp(masked * inv_t - m_all[:, None]), axis=1)
    z = z_edges + jnp.exp(stop * inv_t - m_all)
    log_z = jnp.log(z)

    first_edge = jnp.argmax(masked, axis=1).astype(jnp.int32)
    action = jnp.where(stop >= maxv, 0, first_edge + 1)
    act_ref[:, 0:1] = action[:, None]
    lpf_ref[:, 0:1] = (-log_z)[:, None]
    cm_ref[:, 0:1] = m_all[:, None]
    clz_ref[:, 0:1] = log_z[:, None]
    stop_ref[:, 0:1] = (stop * inv_t - m_all - log_z)[:, None]


def _map_kernel(scores_ref, mask_ref, cm_ref, clz_ref, stop_ref, temp_ref,
                out_ref, carry_ref, *, c):
    j = pl.program_id(0)
    t = jnp.maximum(temp_ref[0, 0], jnp.float32(MIN_TEMPERATURE))
    inv_t = 1.0 / t
    cm = cm_ref[:, 0:1]
    clz = clz_ref[:, 0:1]

    scores = scores_ref[:, :]
    mask = mask_ref[:, :] != 0
    lp = (jnp.where(mask, scores, jnp.float32(NEG)) * inv_t - cm) - clz
    lp_t = jnp.swapaxes(lp, 0, 1)

    stop_t = jnp.swapaxes(stop_ref[:, 0:1], 0, 1)
    head = jnp.where(j == 0, stop_t, carry_ref[:, :])
    carry_ref[:, :] = lp_t[c - 1:c, :]
    out_ref[:, :] = jnp.concatenate([head, lp_t[:c - 1, :]], axis=0)


def kernel(edge_scores, edge_valid_mask, hidden, ln_w, ln_b, W_stop, b_stop,
           temp):
    b, n = edge_scores.shape
    h = hidden.shape[1]
    rows = 32

    mask8 = edge_valid_mask.astype(jnp.int8)
    lnw2 = ln_w.reshape(1, h + 2)
    lnb2 = ln_b.reshape(1, h + 2)
    w2 = W_stop.reshape(1, h + 2)
    bstop2 = b_stop.reshape(1, 1)
    temp2 = temp.reshape(1, 1)

    act, lpf, cm, clz, stop_lp = pl.pallas_call(
        functools.partial(_reduce_kernel, n=n, h=h, rows=rows),
        grid=(b // rows,),
        in_specs=[
            pl.BlockSpec((rows, n), lambda i: (i, 0)),
            pl.BlockSpec((rows, n), lambda i: (i, 0)),
            pl.BlockSpec((rows, h), lambda i: (i, 0)),
            pl.BlockSpec((1, h + 2), lambda i: (0, 0)),
            pl.BlockSpec((1, h + 2), lambda i: (0, 0)),
            pl.BlockSpec((1, h + 2), lambda i: (0, 0)),
            pl.BlockSpec((1, 1), lambda i: (0, 0)),
            pl.BlockSpec((1, 1), lambda i: (0, 0)),
        ],
        out_specs=[
            pl.BlockSpec((rows, 1), lambda i: (i, 0)),
            pl.BlockSpec((rows, 1), lambda i: (i, 0)),
            pl.BlockSpec((rows, 1), lambda i: (i, 0)),
            pl.BlockSpec((rows, 1), lambda i: (i, 0)),
            pl.BlockSpec((rows, 1), lambda i: (i, 0)),
        ],
        out_shape=[
            jax.ShapeDtypeStruct((b, 1), jnp.int32),
            jax.ShapeDtypeStruct((b, 1), jnp.float32),
            jax.ShapeDtypeStruct((b, 1), jnp.float32),
            jax.ShapeDtypeStruct((b, 1), jnp.float32),
            jax.ShapeDtypeStruct((b, 1), jnp.float32),
        ],
        compiler_params=pltpu.CompilerParams(
            dimension_semantics=("parallel",)),
    )(edge_scores, mask8, hidden, lnw2, lnb2, w2, bstop2, temp2)

    c = 2048
    nblocks = (n + 1 + c - 1) // c
    nb = n // c

    lp_t = pl.pallas_call(
        functools.partial(_map_kernel, c=c),
        grid=(nblocks,),
        in_specs=[
            pl.BlockSpec((b, c), lambda j: (0, jnp.minimum(j, nb - 1))),
            pl.BlockSpec((b, c), lambda j: (0, jnp.minimum(j, nb - 1))),
            pl.BlockSpec((b, 1), lambda j: (0, 0)),
            pl.BlockSpec((b, 1), lambda j: (0, 0)),
            pl.BlockSpec((b, 1), lambda j: (0, 0)),
            pl.BlockSpec((1, 1), lambda j: (0, 0)),
        ],
        out_specs=pl.BlockSpec((c, b), lambda j: (j, 0)),
        out_shape=jax.ShapeDtypeStruct((n + 1, b), jnp.float32),
        scratch_shapes=[pltpu.VMEM((1, b), jnp.float32)],
        compiler_params=pltpu.CompilerParams(
            dimension_semantics=("arbitrary",)),
    )(edge_scores, mask8, cm, clz, stop_lp, temp2)

    return act[:, 0], lpf[:, 0], lp_t.T


# speedup vs baseline: 1.1185x; 1.1185x over previous
import functools

import jax
import jax.numpy as jnp
from jax.experimental import pallas as pl
from jax.experimental.pallas import tpu as pltpu

MIN_TEMPERATURE = 1e-05
NEG = -1e9


def _reduce_kernel(scores_ref, mask_ref, hidden_ref, lnw_ref, lnb_ref,
                   w_ref, bstop_ref, temp_ref,
                   act_ref, lpf_ref, cm_ref, clz_ref, stop_ref, *, n, h,
                   rows):
    i = pl.program_id(0)
    scores = scores_ref[:, :]
    mask = mask_ref[:, :] != 0

    masked = jnp.where(mask, scores, jnp.float32(NEG))
    maxv = jnp.max(masked, axis=1)
    has_edge = maxv != jnp.float32(NEG)
    has_f = has_edge.astype(jnp.float32)
    mes = jnp.where(has_edge, maxv, jnp.float32(0.0))

    lnw = lnw_ref[0, :]
    lnb = lnb_ref[0, :]
    w = w_ref[0, :]
    a_full = lnw * w
    a = a_full[:h]
    s_mes = a_full[h]
    s_he = a_full[h + 1]
    c0 = jnp.sum(lnb * w) + bstop_ref[0, 0]
    t = jnp.maximum(temp_ref[0, 0], jnp.float32(MIN_TEMPERATURE))
    inv_t = 1.0 / t

    hid = hidden_ref[:, :]
    denom = jnp.float32(h + 2)
    mu = (jnp.sum(hid, axis=1) + mes + has_f) / denom
    dh = hid - mu[:, None]
    var = (jnp.sum(dh * dh, axis=1) + jnp.square(mes - mu)
           + jnp.square(has_f - mu)) / denom
    inv_std = jax.lax.rsqrt(var + jnp.float32(1e-5))

    dot = jnp.sum(dh * a[None, :], axis=1)
    stop = inv_std * (dot + (mes - mu) * s_mes + (has_f - mu) * s_he) + c0

    m_all = jnp.maximum(stop, maxv) * inv_t
    z_edges = jnp.sum(jnp.exp(masked * inv_t - m_all[:, None]), axis=1)
    z = z_edges + jnp.exp(stop * inv_t - m_all)
    log_z = jnp.log(z)

    first_edge = jnp.argmax(masked, axis=1).astype(jnp.int32)
    action = jnp.where(stop >= maxv, 0, first_edge + 1)
    act_ref[:, 0:1] = action[:, None]
    lpf_ref[:, 0:1] = (-log_z)[:, None]
    cm_ref[:, 0:1] = m_all[:, None]
    clz_ref[:, 0:1] = log_z[:, None]
    stop_ref[:, 0:1] = (stop * inv_t - m_all - log_z)[:, None]


def _map_kernel(scores_ref, mask_ref, cm_ref, clz_ref, stop_ref, temp_ref,
                out_ref, carry_ref, *, c):
    j = pl.program_id(0)
    t = jnp.maximum(temp_ref[0, 0], jnp.float32(MIN_TEMPERATURE))
    inv_t = 1.0 / t
    cm = cm_ref[:, 0:1]
    clz = clz_ref[:, 0:1]

    scores = scores_ref[:, :]
    mask = mask_ref[:, :] != 0
    lp = (jnp.where(mask, scores, jnp.float32(NEG)) * inv_t - cm) - clz
    lp_t = jnp.swapaxes(lp, 0, 1)

    stop_t = jnp.swapaxes(stop_ref[:, 0:1], 0, 1)
    head = jnp.where(j == 0, stop_t, carry_ref[:, :])
    carry_ref[:, :] = lp_t[c - 1:c, :]
    out_ref[:, :] = jnp.concatenate([head, lp_t[:c - 1, :]], axis=0)


def kernel(edge_scores, edge_valid_mask, hidden, ln_w, ln_b, W_stop, b_stop,
           temp):
    b, n = edge_scores.shape
    h = hidden.shape[1]
    rows = 32

    mask8 = edge_valid_mask.astype(jnp.int8)
    lnw2 = ln_w.reshape(1, h + 2)
    lnb2 = ln_b.reshape(1, h + 2)
    w2 = W_stop.reshape(1, h + 2)
    bstop2 = b_stop.reshape(1, 1)
    temp2 = temp.reshape(1, 1)

    act, lpf, cm, clz, stop_lp = pl.pallas_call(
        functools.partial(_reduce_kernel, n=n, h=h, rows=rows),
        grid=(b // rows,),
        in_specs=[
            pl.BlockSpec((rows, n), lambda i: (i, 0)),
            pl.BlockSpec((rows, n), lambda i: (i, 0)),
            pl.BlockSpec((rows, h), lambda i: (i, 0)),
            pl.BlockSpec((1, h + 2), lambda i: (0, 0)),
            pl.BlockSpec((1, h + 2), lambda i: (0, 0)),
            pl.BlockSpec((1, h + 2), lambda i: (0, 0)),
            pl.BlockSpec((1, 1), lambda i: (0, 0)),
            pl.BlockSpec((1, 1), lambda i: (0, 0)),
        ],
        out_specs=[
            pl.BlockSpec((rows, 1), lambda i: (i, 0)),
            pl.BlockSpec((rows, 1), lambda i: (i, 0)),
            pl.BlockSpec((rows, 1), lambda i: (i, 0)),
            pl.BlockSpec((rows, 1), lambda i: (i, 0)),
            pl.BlockSpec((rows, 1), lambda i: (i, 0)),
        ],
        out_shape=[
            jax.ShapeDtypeStruct((b, 1), jnp.int32),
            jax.ShapeDtypeStruct((b, 1), jnp.float32),
            jax.ShapeDtypeStruct((b, 1), jnp.float32),
            jax.ShapeDtypeStruct((b, 1), jnp.float32),
            jax.ShapeDtypeStruct((b, 1), jnp.float32),
        ],
        compiler_params=pltpu.CompilerParams(
            dimension_semantics=("parallel",)),
    )(edge_scores, mask8, hidden, lnw2, lnb2, w2, bstop2, temp2)

    c = 4096
    nblocks = (n + 1 + c - 1) // c
    nb = n // c

    lp_t = pl.pallas_call(
        functools.partial(_map_kernel, c=c),
        grid=(nblocks,),
        in_specs=[
            pl.BlockSpec((b, c), lambda j: (0, jnp.minimum(j, nb - 1))),
            pl.BlockSpec((b, c), lambda j: (0, jnp.minimum(j, nb - 1))),
            pl.BlockSpec((b, 1), lambda j: (0, 0)),
            pl.BlockSpec((b, 1), lambda j: (0, 0)),
            pl.BlockSpec((b, 1), lambda j: (0, 0)),
            pl.BlockSpec((1, 1), lambda j: (0, 0)),
        ],
        out_specs=pl.BlockSpec((c, b), lambda j: (j, 0)),
        out_shape=jax.ShapeDtypeStruct((n + 1, b), jnp.float32),
        scratch_shapes=[pltpu.VMEM((1, b), jnp.float32)],
        compiler_params=pltpu.CompilerParams(
            dimension_semantics=("arbitrary",)),
    )(edge_scores, mask8, cm, clz, stop_lp, temp2)

    return act[:, 0], lpf[:, 0], lp_t.T


# speedup vs baseline: 1.1585x; 1.0357x over previous
import functools

import jax
import jax.numpy as jnp
from jax.experimental import pallas as pl
from jax.experimental.pallas import tpu as pltpu

MIN_TEMPERATURE = 1e-05
NEG = -1e9


def _reduce_kernel(scores_ref, mask_ref, hidden_ref, lnw_ref, lnb_ref,
                   w_ref, bstop_ref, temp_ref,
                   act_ref, lpf_ref, cm_ref, clz_ref, stop_ref, *, n, h,
                   rows):
    i = pl.program_id(0)
    scores = scores_ref[:, :]
    mask = mask_ref[:, :] != 0

    masked = jnp.where(mask, scores, jnp.float32(NEG))
    maxv = jnp.max(masked, axis=1)
    has_edge = maxv != jnp.float32(NEG)
    has_f = has_edge.astype(jnp.float32)
    mes = jnp.where(has_edge, maxv, jnp.float32(0.0))

    lnw = lnw_ref[0, :]
    lnb = lnb_ref[0, :]
    w = w_ref[0, :]
    a_full = lnw * w
    a = a_full[:h]
    s_mes = a_full[h]
    s_he = a_full[h + 1]
    c0 = jnp.sum(lnb * w) + bstop_ref[0, 0]
    t = jnp.maximum(temp_ref[0, 0], jnp.float32(MIN_TEMPERATURE))
    inv_t = 1.0 / t

    hid = hidden_ref[:, :]
    denom = jnp.float32(h + 2)
    mu = (jnp.sum(hid, axis=1) + mes + has_f) / denom
    dh = hid - mu[:, None]
    var = (jnp.sum(dh * dh, axis=1) + jnp.square(mes - mu)
           + jnp.square(has_f - mu)) / denom
    inv_std = jax.lax.rsqrt(var + jnp.float32(1e-5))

    dot = jnp.sum(dh * a[None, :], axis=1)
    stop = inv_std * (dot + (mes - mu) * s_mes + (has_f - mu) * s_he) + c0

    m_all = jnp.maximum(stop, maxv) * inv_t
    z_edges = jnp.sum(jnp.exp(masked * inv_t - m_all[:, None]), axis=1)
    z = z_edges + jnp.exp(stop * inv_t - m_all)
    log_z = jnp.log(z)

    first_edge = jnp.argmax(masked, axis=1).astype(jnp.int32)
    action = jnp.where(stop >= maxv, 0, first_edge + 1)
    act_ref[:, 0:1] = action[:, None]
    lpf_ref[:, 0:1] = (-log_z)[:, None]
    cm_ref[:, 0:1] = m_all[:, None]
    clz_ref[:, 0:1] = log_z[:, None]
    stop_ref[:, 0:1] = (stop * inv_t - m_all - log_z)[:, None]


def _map_kernel(scores_ref, mask_ref, cm_ref, clz_ref, stop_ref, temp_ref,
                out_ref, carry_ref, *, c):
    j = pl.program_id(0)
    t = jnp.maximum(temp_ref[0, 0], jnp.float32(MIN_TEMPERATURE))
    inv_t = 1.0 / t
    cm = cm_ref[:, 0:1]
    clz = clz_ref[:, 0:1]

    scores = scores_ref[:, :]
    mask = mask_ref[:, :] != 0
    lp = (jnp.where(mask, scores, jnp.float32(NEG)) * inv_t - cm) - clz
    lp_t = jnp.swapaxes(lp, 0, 1)

    stop_t = jnp.swapaxes(stop_ref[:, 0:1], 0, 1)
    head = jnp.where(j == 0, stop_t, carry_ref[:, :])
    carry_ref[:, :] = lp_t[c - 1:c, :]
    out_ref[:, :] = jnp.concatenate([head, lp_t[:c - 1, :]], axis=0)


def kernel(edge_scores, edge_valid_mask, hidden, ln_w, ln_b, W_stop, b_stop,
           temp):
    b, n = edge_scores.shape
    h = hidden.shape[1]
    rows = 32

    mask8 = edge_valid_mask.astype(jnp.int8)
    lnw2 = ln_w.reshape(1, h + 2)
    lnb2 = ln_b.reshape(1, h + 2)
    w2 = W_stop.reshape(1, h + 2)
    bstop2 = b_stop.reshape(1, 1)
    temp2 = temp.reshape(1, 1)

    act, lpf, cm, clz, stop_lp = pl.pallas_call(
        functools.partial(_reduce_kernel, n=n, h=h, rows=rows),
        grid=(b // rows,),
        in_specs=[
            pl.BlockSpec((rows, n), lambda i: (i, 0)),
            pl.BlockSpec((rows, n), lambda i: (i, 0)),
            pl.BlockSpec((rows, h), lambda i: (i, 0)),
            pl.BlockSpec((1, h + 2), lambda i: (0, 0)),
            pl.BlockSpec((1, h + 2), lambda i: (0, 0)),
            pl.BlockSpec((1, h + 2), lambda i: (0, 0)),
            pl.BlockSpec((1, 1), lambda i: (0, 0)),
            pl.BlockSpec((1, 1), lambda i: (0, 0)),
        ],
        out_specs=[
            pl.BlockSpec((rows, 1), lambda i: (i, 0)),
            pl.BlockSpec((rows, 1), lambda i: (i, 0)),
            pl.BlockSpec((rows, 1), lambda i: (i, 0)),
            pl.BlockSpec((rows, 1), lambda i: (i, 0)),
            pl.BlockSpec((rows, 1), lambda i: (i, 0)),
        ],
        out_shape=[
            jax.ShapeDtypeStruct((b, 1), jnp.int32),
            jax.ShapeDtypeStruct((b, 1), jnp.float32),
            jax.ShapeDtypeStruct((b, 1), jnp.float32),
            jax.ShapeDtypeStruct((b, 1), jnp.float32),
            jax.ShapeDtypeStruct((b, 1), jnp.float32),
        ],
        compiler_params=pltpu.CompilerParams(
            dimension_semantics=("parallel",)),
    )(edge_scores, mask8, hidden, lnw2, lnb2, w2, bstop2, temp2)

    c = 8192
    nblocks = (n + 1 + c - 1) // c
    nb = n // c

    lp_t = pl.pallas_call(
        functools.partial(_map_kernel, c=c),
        grid=(nblocks,),
        in_specs=[
            pl.BlockSpec((b, c), lambda j: (0, jnp.minimum(j, nb - 1))),
            pl.BlockSpec((b, c), lambda j: (0, jnp.minimum(j, nb - 1))),
            pl.BlockSpec((b, 1), lambda j: (0, 0)),
            pl.BlockSpec((b, 1), lambda j: (0, 0)),
            pl.BlockSpec((b, 1), lambda j: (0, 0)),
            pl.BlockSpec((1, 1), lambda j: (0, 0)),
        ],
        out_specs=pl.BlockSpec((c, b), lambda j: (j, 0)),
        out_shape=jax.ShapeDtypeStruct((n + 1, b), jnp.float32),
        scratch_shapes=[pltpu.VMEM((1, b), jnp.float32)],
        compiler_params=pltpu.CompilerParams(
            dimension_semantics=("arbitrary",)),
    )(edge_scores, mask8, cm, clz, stop_lp, temp2)

    return act[:, 0], lpf[:, 0], lp_t.T


# speedup vs baseline: 1.1822x; 1.0205x over previous
import functools

import jax
import jax.numpy as jnp
from jax.experimental import pallas as pl
from jax.experimental.pallas import tpu as pltpu

MIN_TEMPERATURE = 1e-05
NEG = -1e9


def _reduce_kernel(scores_ref, mask_ref, hidden_ref, lnw_ref, lnb_ref,
                   w_ref, bstop_ref, temp_ref,
                   act_ref, lpf_ref, cm_ref, clz_ref, stop_ref, *, n, h,
                   rows):
    i = pl.program_id(0)
    scores = scores_ref[:, :]
    mask = mask_ref[:, :] != 0

    masked = jnp.where(mask, scores, jnp.float32(NEG))
    maxv = jnp.max(masked, axis=1)
    has_edge = maxv != jnp.float32(NEG)
    has_f = has_edge.astype(jnp.float32)
    mes = jnp.where(has_edge, maxv, jnp.float32(0.0))

    lnw = lnw_ref[0, :]
    lnb = lnb_ref[0, :]
    w = w_ref[0, :]
    a_full = lnw * w
    a = a_full[:h]
    s_mes = a_full[h]
    s_he = a_full[h + 1]
    c0 = jnp.sum(lnb * w) + bstop_ref[0, 0]
    t = jnp.maximum(temp_ref[0, 0], jnp.float32(MIN_TEMPERATURE))
    inv_t = 1.0 / t

    hid = hidden_ref[:, :]
    denom = jnp.float32(h + 2)
    mu = (jnp.sum(hid, axis=1) + mes + has_f) / denom
    dh = hid - mu[:, None]
    var = (jnp.sum(dh * dh, axis=1) + jnp.square(mes - mu)
           + jnp.square(has_f - mu)) / denom
    inv_std = jax.lax.rsqrt(var + jnp.float32(1e-5))

    dot = jnp.sum(dh * a[None, :], axis=1)
    stop = inv_std * (dot + (mes - mu) * s_mes + (has_f - mu) * s_he) + c0

    m_all = jnp.maximum(stop, maxv) * inv_t
    z_edges = jnp.sum(jnp.exp(masked * inv_t - m_all[:, None]), axis=1)
    z = z_edges + jnp.exp(stop * inv_t - m_all)
    log_z = jnp.log(z)

    first_edge = jnp.argmax(masked, axis=1).astype(jnp.int32)
    action = jnp.where(stop >= maxv, 0, first_edge + 1)
    act_ref[:, 0:1] = action[:, None]
    lpf_ref[:, 0:1] = (-log_z)[:, None]
    cm_ref[:, 0:1] = m_all[:, None]
    clz_ref[:, 0:1] = log_z[:, None]
    stop_ref[:, 0:1] = (stop * inv_t - m_all - log_z)[:, None]


def _map_kernel(scores_ref, mask_ref, cm_ref, clz_ref, stop_ref, temp_ref,
                out_ref, carry_ref, *, c):
    j = pl.program_id(0)
    t = jnp.maximum(temp_ref[0, 0], jnp.float32(MIN_TEMPERATURE))
    inv_t = 1.0 / t
    cm = cm_ref[:, 0:1]
    clz = clz_ref[:, 0:1]

    scores = scores_ref[:, :]
    mask = mask_ref[:, :] != 0
    lp = (jnp.where(mask, scores, jnp.float32(NEG)) * inv_t - cm) - clz
    lp_t = jnp.swapaxes(lp, 0, 1)

    stop_t = jnp.swapaxes(stop_ref[:, 0:1], 0, 1)
    head = jnp.where(j == 0, stop_t, carry_ref[:, :])
    carry_ref[:, :] = lp_t[c - 1:c, :]
    out_ref[:, :] = jnp.concatenate([head, lp_t[:c - 1, :]], axis=0)


def kernel(edge_scores, edge_valid_mask, hidden, ln_w, ln_b, W_stop, b_stop,
           temp):
    b, n = edge_scores.shape
    h = hidden.shape[1]
    rows = 32

    mask8 = edge_valid_mask.astype(jnp.int8)
    lnw2 = ln_w.reshape(1, h + 2)
    lnb2 = ln_b.reshape(1, h + 2)
    w2 = W_stop.reshape(1, h + 2)
    bstop2 = b_stop.reshape(1, 1)
    temp2 = temp.reshape(1, 1)

    act, lpf, cm, clz, stop_lp = pl.pallas_call(
        functools.partial(_reduce_kernel, n=n, h=h, rows=rows),
        grid=(b // rows,),
        in_specs=[
            pl.BlockSpec((rows, n), lambda i: (i, 0)),
            pl.BlockSpec((rows, n), lambda i: (i, 0)),
            pl.BlockSpec((rows, h), lambda i: (i, 0)),
            pl.BlockSpec((1, h + 2), lambda i: (0, 0)),
            pl.BlockSpec((1, h + 2), lambda i: (0, 0)),
            pl.BlockSpec((1, h + 2), lambda i: (0, 0)),
            pl.BlockSpec((1, 1), lambda i: (0, 0)),
            pl.BlockSpec((1, 1), lambda i: (0, 0)),
        ],
        out_specs=[
            pl.BlockSpec((rows, 1), lambda i: (i, 0)),
            pl.BlockSpec((rows, 1), lambda i: (i, 0)),
            pl.BlockSpec((rows, 1), lambda i: (i, 0)),
            pl.BlockSpec((rows, 1), lambda i: (i, 0)),
            pl.BlockSpec((rows, 1), lambda i: (i, 0)),
        ],
        out_shape=[
            jax.ShapeDtypeStruct((b, 1), jnp.int32),
            jax.ShapeDtypeStruct((b, 1), jnp.float32),
            jax.ShapeDtypeStruct((b, 1), jnp.float32),
            jax.ShapeDtypeStruct((b, 1), jnp.float32),
            jax.ShapeDtypeStruct((b, 1), jnp.float32),
        ],
        compiler_params=pltpu.CompilerParams(
            dimension_semantics=("parallel",)),
    )(edge_scores, mask8, hidden, lnw2, lnb2, w2, bstop2, temp2)

    c = 16384
    nblocks = (n + 1 + c - 1) // c
    nb = n // c

    lp_t = pl.pallas_call(
        functools.partial(_map_kernel, c=c),
        grid=(nblocks,),
        in_specs=[
            pl.BlockSpec((b, c), lambda j: (0, jnp.minimum(j, nb - 1))),
            pl.BlockSpec((b, c), lambda j: (0, jnp.minimum(j, nb - 1))),
            pl.BlockSpec((b, 1), lambda j: (0, 0)),
            pl.BlockSpec((b, 1), lambda j: (0, 0)),
            pl.BlockSpec((b, 1), lambda j: (0, 0)),
            pl.BlockSpec((1, 1), lambda j: (0, 0)),
        ],
        out_specs=pl.BlockSpec((c, b), lambda j: (j, 0)),
        out_shape=jax.ShapeDtypeStruct((n + 1, b), jnp.float32),
        scratch_shapes=[pltpu.VMEM((1, b), jnp.float32)],
        compiler_params=pltpu.CompilerParams(
            dimension_semantics=("arbitrary",)),
    )(edge_scores, mask8, cm, clz, stop_lp, temp2)

    return act[:, 0], lpf[:, 0], lp_t.T
